# Initial kernel scaffold; baseline (speedup 1.0000x reference)
import functools

import jax
import jax.numpy as jnp
from jax import lax
from jax.experimental import pallas as pl
from jax.experimental.pallas import tpu as pltpu
from jax.experimental.pallas import tpu_sc as plsc

N = 10000
E = 320000
D = 128
EPS = 1e-5
SLOPE = 0.01

NC = 2
NS = 16
NW = NC * NS
B = 80
CH = E // (NW * B)
RPT = N // NS

_mesh = plsc.VectorSubcoreMesh(core_axis_name="c", subcore_axis_name="s")


@functools.partial(
    pl.kernel,
    out_type=jax.ShapeDtypeStruct((NC, N, 16), jnp.float32),
    mesh=_mesh,
    scratch_types=[
        pltpu.VMEM_SHARED((N, 16), jnp.float32),
        pltpu.VMEM((CH, B), jnp.int32),
        pltpu.VMEM((B, 16), jnp.float32),
    ],
)
def _deg_kernel(dst_hbm, ones_hbm, zeros_hbm, out_hbm, deg_sh, idx_v, ones_v):
    c = lax.axis_index("c")
    s = lax.axis_index("s")
    wid = s * NC + c
    pltpu.sync_copy(zeros_hbm, deg_sh.at[pl.ds(s * RPT, RPT)])
    pltpu.sync_copy(ones_hbm, ones_v)
    pltpu.sync_copy(dst_hbm.at[pl.ds(wid * CH, CH)], idx_v)
    plsc.subcore_barrier()

    def body(g, carry):
        pltpu.sync_copy(ones_v, deg_sh.at[idx_v.at[g]], add=True)
        return carry

    lax.fori_loop(0, CH, body, 0)
    plsc.subcore_barrier()
    pltpu.sync_copy(deg_sh.at[pl.ds(s * RPT, RPT)],
                    out_hbm.at[c, pl.ds(s * RPT, RPT)])


@functools.partial(
    pl.kernel,
    out_type=jax.ShapeDtypeStruct((NC, N, D), jnp.float32),
    mesh=_mesh,
    scratch_types=[
        pltpu.VMEM_SHARED((N, D), jnp.float32),
        pltpu.VMEM((CH, B), jnp.int32),
        pltpu.VMEM((CH, B), jnp.int32),
        pltpu.VMEM((B, D), jnp.float32),
        pltpu.SemaphoreType.DMA,
    ],
)
def _agg_kernel(hs_hbm, src_hbm, dst_hbm, zeros_hbm, out_hbm,
                agg_sh, sidx, didx, rows, sem):
    c = lax.axis_index("c")
    s = lax.axis_index("s")
    wid = s * NC + c
    pltpu.sync_copy(zeros_hbm, agg_sh.at[pl.ds(s * RPT, RPT)])
    pltpu.sync_copy(src_hbm.at[pl.ds(wid * CH, CH)], sidx)
    pltpu.sync_copy(dst_hbm.at[pl.ds(wid * CH, CH)], didx)
    plsc.subcore_barrier()

    def body(g, carry):
        pltpu.async_copy(hs_hbm.at[sidx.at[g]], rows, sem).wait()
        pltpu.sync_copy(rows, agg_sh.at[didx.at[g]], add=True)
        return carry

    lax.fori_loop(0, CH, body, 0)
    plsc.subcore_barrier()
    pltpu.sync_copy(agg_sh.at[pl.ds(s * RPT, RPT)],
                    out_hbm.at[c, pl.ds(s * RPT, RPT)])


def _leaky(x):
    return jnp.where(x > 0, x, SLOPE * x)


def _pool_max(y, pad):
    acc = y
    ninf = jnp.float32(-jnp.inf)
    iota = lax.broadcasted_iota(jnp.int32, y.shape, 1)
    for sft in range(1, pad + 1):
        t = jnp.where(iota >= sft, pltpu.roll(y, sft, 1), ninf)
        acc = jnp.maximum(acc, t)
        t = jnp.where(iota < D - sft, pltpu.roll(y, -sft, 1), ninf)
        acc = jnp.maximum(acc, t)
    return acc


def _first_body(cnt_ref, x_ref, w_ref, dinv_ref, hs_ref):
    cnt = cnt_ref[0][:, 0:1] + cnt_ref[1][:, 0:1]
    dinv = lax.rsqrt(cnt + 2.0)
    dinv_ref[...] = dinv
    h = jnp.dot(x_ref[...], w_ref[...], preferred_element_type=jnp.float32)
    hs_ref[...] = h * dinv


_tc_first = pl.pallas_call(
    _first_body,
    out_shape=[jax.ShapeDtypeStruct((N, 1), jnp.float32),
               jax.ShapeDtypeStruct((N, D), jnp.float32)],
)


def _block_body(p_ref, hs_ref, dinv_ref, b_ref, g_ref, be_ref, lwT_ref,
                lb_ref, wn_ref, out_ref, *, pad, last):
    dinv = dinv_ref[...]
    pre = dinv * (p_ref[0] + p_ref[1] + 2.0 * hs_ref[...]) + b_ref[...]
    m = jnp.mean(pre, axis=0, keepdims=True)
    v = jnp.mean(pre * pre, axis=0, keepdims=True) - m * m
    y = g_ref[...] * ((pre - m) * lax.rsqrt(v + EPS)) + be_ref[...]
    y = _leaky(_pool_max(y, pad))
    z = jnp.dot(y, lwT_ref[...], preferred_element_type=jnp.float32)
    z = _leaky(z + lb_ref[...])
    if last:
        out_ref[...] = z
    else:
        out_ref[...] = dinv * jnp.dot(z, wn_ref[...],
                                      preferred_element_type=jnp.float32)


def _tc_block(pad, last):
    return pl.pallas_call(
        functools.partial(_block_body, pad=pad, last=last),
        out_shape=jax.ShapeDtypeStruct((N, D), jnp.float32),
    )


_tc_b0 = _tc_block(1, False)
_tc_b1 = _tc_block(2, False)
_tc_b2 = _tc_block(1, True)


def kernel(x, edge_index, W0, b0, g0, be0, lw0, lb0,
           W1, b1, g1, be1, lw1, lb1, W2, b2, g2, be2, lw2, lb2):
    src = edge_index[0].reshape(E // B, B)
    dst = edge_index[1].reshape(E // B, B)
    ones16 = jnp.ones((B, 16), jnp.float32)
    zeros16 = jnp.zeros((RPT, 16), jnp.float32)
    zeros128 = jnp.zeros((RPT, D), jnp.float32)
    row = lambda v: v.reshape(1, D)

    cnt = _deg_kernel(dst, ones16, zeros16)
    dinv, hs = _tc_first(cnt, x, W0)

    p = _agg_kernel(hs, src, dst, zeros128)
    hs = _tc_b0(p, hs, dinv, row(b0), row(g0), row(be0), lw0.T, row(lb0), W1)

    p = _agg_kernel(hs, src, dst, zeros128)
    hs = _tc_b1(p, hs, dinv, row(b1), row(g1), row(be1), lw1.T, row(lb1), W2)

    p = _agg_kernel(hs, src, dst, zeros128)
    out = _tc_b2(p, hs, dinv, row(b2), row(g2), row(be2), lw2.T, row(lb2), W2)
    return out


# speedup vs baseline: 16.2803x; 16.2803x over previous
import functools

import jax
import jax.numpy as jnp
from jax import lax
from jax.experimental import pallas as pl
from jax.experimental.pallas import tpu as pltpu
from jax.experimental.pallas import tpu_sc as plsc

N = 10000
E = 320000
D = 128
EPS = 1e-5
SLOPE = 0.01

NC = 2
NS = 16
NW = NC * NS
B = 80
CH = E // (NW * B)
RPT = 632
N_PAD = RPT * NS


@functools.cache
def _sc_kernels():
    mesh = plsc.VectorSubcoreMesh(core_axis_name="c", subcore_axis_name="s",
                                  num_cores=NC, num_subcores=NS)

    @functools.partial(
        pl.kernel,
        out_type=jax.ShapeDtypeStruct((NC, N_PAD, 16), jnp.float32),
        mesh=mesh,
        scratch_types=[
            pltpu.VMEM_SHARED((N_PAD, 16), jnp.float32),
            pltpu.VMEM((CH, B), jnp.int32),
            pltpu.VMEM((B, 16), jnp.float32),
        ],
    )
    def deg_kernel(dst_hbm, ones_hbm, zeros_hbm, out_hbm, deg_sh, idx_v, ones_v):
        c = lax.axis_index("c")
        s = lax.axis_index("s")
        wid = s * NC + c
        pltpu.sync_copy(zeros_hbm, deg_sh.at[pl.ds(s * RPT, RPT)])
        pltpu.sync_copy(ones_hbm, ones_v)
        pltpu.sync_copy(dst_hbm.at[wid], idx_v)
        plsc.subcore_barrier()

        def body(g, carry):
            pltpu.sync_copy(ones_v, deg_sh.at[idx_v.at[g]], add=True)
            return carry

        lax.fori_loop(0, CH, body, 0)
        plsc.subcore_barrier()
        pltpu.sync_copy(deg_sh.at[pl.ds(s * RPT, RPT)],
                        out_hbm.at[c, pl.ds(s * RPT, RPT)])

    @functools.partial(
        pl.kernel,
        out_type=jax.ShapeDtypeStruct((NC, N_PAD, D), jnp.float32),
        mesh=mesh,
        scratch_types=[
            pltpu.VMEM_SHARED((N_PAD, D), jnp.float32),
            pltpu.VMEM((CH, B), jnp.int32),
            pltpu.VMEM((CH, B), jnp.int32),
            pltpu.VMEM((B, D), jnp.float32),
            pltpu.SemaphoreType.DMA,
        ],
    )
    def agg_kernel(hs_hbm, src_hbm, dst_hbm, zeros_hbm, out_hbm,
                   agg_sh, sidx, didx, rows, sem):
        c = lax.axis_index("c")
        s = lax.axis_index("s")
        wid = s * NC + c
        pltpu.sync_copy(zeros_hbm, agg_sh.at[pl.ds(s * RPT, RPT)])
        pltpu.sync_copy(src_hbm.at[wid], sidx)
        pltpu.sync_copy(dst_hbm.at[wid], didx)
        plsc.subcore_barrier()

        def body(g, carry):
            pltpu.async_copy(hs_hbm.at[sidx.at[g]], rows, sem).wait()
            pltpu.sync_copy(rows, agg_sh.at[didx.at[g]], add=True)
            return carry

        lax.fori_loop(0, CH, body, 0)
        plsc.subcore_barrier()
        pltpu.sync_copy(agg_sh.at[pl.ds(s * RPT, RPT)],
                        out_hbm.at[c, pl.ds(s * RPT, RPT)])

    return deg_kernel, agg_kernel


def _leaky(x):
    return jnp.where(x > 0, x, SLOPE * x)


def _pool_max(y, pad):
    acc = y
    for sft in range(1, pad + 1):
        fill = jnp.full((y.shape[0], sft), -jnp.inf, y.dtype)
        acc = jnp.maximum(acc, jnp.concatenate([fill, y[:, :D - sft]], axis=1))
        acc = jnp.maximum(acc, jnp.concatenate([y[:, sft:], fill], axis=1))
    return acc


def _first_body(cnt_ref, x_ref, w_ref, dinv_ref, hs_ref):
    cnt = cnt_ref[0][:N, 0:1] + cnt_ref[1][:N, 0:1]
    dinv = lax.rsqrt(cnt + 2.0)
    dinv_ref[...] = dinv
    h = jnp.dot(x_ref[...], w_ref[...], preferred_element_type=jnp.float32)
    hs_ref[...] = h * dinv


_tc_first = pl.pallas_call(
    _first_body,
    out_shape=[jax.ShapeDtypeStruct((N, 1), jnp.float32),
               jax.ShapeDtypeStruct((N, D), jnp.float32)],
)


def _block_body(p_ref, hs_ref, dinv_ref, b_ref, g_ref, be_ref, lwT_ref,
                lb_ref, wn_ref, out_ref, *, pad, last):
    dinv = dinv_ref[...]
    pre = dinv * (p_ref[0][:N] + p_ref[1][:N] + 2.0 * hs_ref[...]) + b_ref[...]
    m = jnp.mean(pre, axis=0, keepdims=True)
    v = jnp.mean(pre * pre, axis=0, keepdims=True) - m * m
    y = g_ref[...] * ((pre - m) * lax.rsqrt(v + EPS)) + be_ref[...]
    y = _leaky(_pool_max(y, pad))
    z = jnp.dot(y, lwT_ref[...], preferred_element_type=jnp.float32)
    z = _leaky(z + lb_ref[...])
    if last:
        out_ref[...] = z
    else:
        out_ref[...] = dinv * jnp.dot(z, wn_ref[...],
                                      preferred_element_type=jnp.float32)


def _tc_block(pad, last):
    return pl.pallas_call(
        functools.partial(_block_body, pad=pad, last=last),
        out_shape=jax.ShapeDtypeStruct((N, D), jnp.float32),
    )


_tc_b0 = _tc_block(1, False)
_tc_b1 = _tc_block(2, False)
_tc_b2 = _tc_block(1, True)


def kernel(x, edge_index, W0, b0, g0, be0, lw0, lb0,
           W1, b1, g1, be1, lw1, lb1, W2, b2, g2, be2, lw2, lb2):
    src = edge_index[0].reshape(NW, CH, B)
    dst = edge_index[1].reshape(NW, CH, B)
    ones16 = jnp.ones((B, 16), jnp.float32)
    zeros16 = jnp.zeros((RPT, 16), jnp.float32)
    zeros128 = jnp.zeros((RPT, D), jnp.float32)
    row = lambda v: v.reshape(1, D)

    deg_kernel, agg_kernel = _sc_kernels()
    cnt = deg_kernel(dst, ones16, zeros16)
    dinv, hs = _tc_first(cnt, x, W0)

    p = agg_kernel(hs, src, dst, zeros128)
    hs = _tc_b0(p, hs, dinv, row(b0), row(g0), row(be0), lw0.T, row(lb0), W1)

    p = agg_kernel(hs, src, dst, zeros128)
    hs = _tc_b1(p, hs, dinv, row(b1), row(g1), row(be1), lw1.T, row(lb1), W2)

    p = agg_kernel(hs, src, dst, zeros128)
    out = _tc_b2(p, hs, dinv, row(b2), row(g2), row(be2), lw2.T, row(lb2), W2)
    return out
